# Initial kernel scaffold; baseline (speedup 1.0000x reference)
import jax
import jax.numpy as jnp
from jax.experimental import pallas as pl


def kernel(x, edge_index, eps1, W1a, b1a, g1, be1, W1b, b1b, eps2, W2a, b2a, g2, be2, W2b, b2b):
    raise NotImplementedError("write your pallas kernel here")


# speedup vs baseline: 5.0843x; 5.0843x over previous
import functools

import jax
import jax.numpy as jnp
from jax import lax
from jax.experimental import pallas as pl
from jax.experimental.pallas import tpu as pltpu
from jax.experimental.pallas import tpu_sc as plsc

_N = 10000
_E = 320000
_D = 128
_H = 128
_C = 64

_NC = 2
_NS = 16
_NW = _NC * _NS
_EPW = _E // _NW
_B = 80
_NITER = _EPW // _B
_RPS = 632
_NPAD = _RPS * _NS


def _sc_agg_body(x_hbm, src_hbm, dst_hbm, zeros_hbm, out_hbm,
                 src_v, dst_v, rows_v, agg_sh, sem):
    c = lax.axis_index("c")
    s = lax.axis_index("s")
    w = c * _NS + s

    pltpu.sync_copy(zeros_hbm.at[pl.ds(s * _RPS, _RPS)],
                    agg_sh.at[pl.ds(s * _RPS, _RPS)])
    plsc.subcore_barrier()

    def step(j, carry):
        base = w * _EPW + j * _B
        pltpu.sync_copy(src_hbm.at[pl.ds(base, _B)], src_v)
        pltpu.sync_copy(dst_hbm.at[pl.ds(base, _B)], dst_v)
        pltpu.async_copy(x_hbm.at[src_v], rows_v, sem).wait()
        pltpu.sync_copy(rows_v, agg_sh.at[dst_v], add=True)
        return carry

    lax.fori_loop(0, _NITER, step, 0)

    plsc.subcore_barrier()
    pltpu.sync_copy(agg_sh.at[pl.ds(s * _RPS, _RPS)],
                    out_hbm.at[c, pl.ds(s * _RPS, _RPS)])


@jax.jit
def _sc_segment_sum(x, src, dst, zeros):
    mesh = plsc.VectorSubcoreMesh(core_axis_name="c", subcore_axis_name="s")
    f = pl.kernel(
        _sc_agg_body,
        out_type=jax.ShapeDtypeStruct((_NC, _NPAD, _D), jnp.float32),
        mesh=mesh,
        scratch_types=[
            pltpu.VMEM((_B,), jnp.int32),
            pltpu.VMEM((_B,), jnp.int32),
            pltpu.VMEM((_B, _D), jnp.float32),
            pltpu.VMEM_SHARED((_NPAD, _D), jnp.float32),
            pltpu.SemaphoreType.DMA,
        ],
    )
    return f(x, src, dst, zeros)


def _mlp_body(eps_ref, x_ref, agg_ref, wa_ref, ba_ref, g_ref, be_ref,
              wb_ref, bb_ref, o_ref, *, last):
    agg = agg_ref[0, :_N, :] + agg_ref[1, :_N, :]
    h = (1.0 + eps_ref[0]) * x_ref[...] + agg
    t = jnp.dot(h, wa_ref[...], preferred_element_type=jnp.float32) + ba_ref[...]
    t = jnp.maximum(t, 0.0)
    mu = jnp.mean(t, axis=0, keepdims=True)
    var = jnp.mean((t - mu) ** 2, axis=0, keepdims=True)
    t = g_ref[...] * (t - mu) * lax.rsqrt(var + 1e-5) + be_ref[...]
    o = jnp.dot(t, wb_ref[...], preferred_element_type=jnp.float32) + bb_ref[...]
    if last:
        o = o - jnp.max(o, axis=-1, keepdims=True)
        o = o - jnp.log(jnp.sum(jnp.exp(o), axis=-1, keepdims=True))
    else:
        o = jnp.maximum(o, 0.0)
    o_ref[...] = o


def _mlp(eps, x, agg, wa, ba, g, be, wb, bb, *, last):
    cout = wb.shape[1]
    return pl.pallas_call(
        functools.partial(_mlp_body, last=last),
        out_shape=jax.ShapeDtypeStruct((_N, cout), jnp.float32),
        in_specs=[pl.BlockSpec(memory_space=pltpu.SMEM)]
        + [pl.BlockSpec(memory_space=pltpu.VMEM)] * 8,
        out_specs=pl.BlockSpec(memory_space=pltpu.VMEM),
    )(eps, x, agg, wa, ba, g, be, wb, bb)


def kernel(x, edge_index, eps1, W1a, b1a, g1, be1, W1b, b1b,
           eps2, W2a, b2a, g2, be2, W2b, b2b):
    ei = edge_index.astype(jnp.int32)
    zeros = jnp.zeros((_NPAD, _D), jnp.float32)
    e1 = jnp.reshape(eps1, (1,)).astype(jnp.float32)
    e2 = jnp.reshape(eps2, (1,)).astype(jnp.float32)

    src, dst = ei[0], ei[1]
    agg1 = _sc_segment_sum(x, src, dst, zeros)
    h1 = _mlp(e1, x, agg1, W1a, b1a.reshape(1, _H), g1.reshape(1, _H),
              be1.reshape(1, _H), W1b, b1b.reshape(1, _H), last=False)
    agg2 = _sc_segment_sum(h1, src, dst, zeros)
    out = _mlp(e2, h1, agg2, W2a, b2a.reshape(1, _H), g2.reshape(1, _H),
               be2.reshape(1, _H), W2b, b2b.reshape(1, _C), last=True)
    return out
